# baseline (speedup 1.0000x reference)
import jax
import jax.numpy as jnp
from jax import lax
from jax.experimental import pallas as pl
from jax.experimental.pallas import tpu as pltpu
from jax.experimental.pallas import tpu_sc as plsc

N = 10000
E = 320000
NC = 2
NS = 16
NW = NC * NS
K = 128
KM = 80
EPT = E // NW
NCHUNK = 80
NCHUNKM = EPT // KM
EPTP = NCHUNK * K
NPAD = 10112
RPT = NPAD // NS
D = 128
BM = 400
F32 = jnp.float32

_MESH = plsc.VectorSubcoreMesh(core_axis_name="c", subcore_axis_name="s")


def _deg_body(src_hbm, dst_hbm, out_hbm, sidx_all, didx_all, ones_v, zrow_v,
              acc_out, acc_in, sem0, sem1):
    c = lax.axis_index("c")
    s = lax.axis_index("s")
    wid = s * NC + c

    @pl.loop(0, K, step=16)
    def _(i):
        ones_v[pl.ds(i, 16)] = jnp.ones((16,), F32)

    @pl.loop(0, RPT + 8, step=16)
    def _(i):
        zrow_v[pl.ds(i, 16)] = jnp.zeros((16,), F32)

    pltpu.sync_copy(src_hbm.at[wid], sidx_all)
    pltpu.sync_copy(dst_hbm.at[wid], didx_all)

    r0 = s * RPT
    pltpu.sync_copy(zrow_v.at[pl.ds(0, RPT)], acc_out.at[pl.ds(r0, RPT)])
    pltpu.sync_copy(zrow_v.at[pl.ds(0, RPT)], acc_in.at[pl.ds(r0, RPT)])
    plsc.subcore_barrier()

    def fire(j):
        pltpu.async_copy(ones_v, acc_out.at[sidx_all.at[j]], sem0, add=True)
        pltpu.async_copy(ones_v, acc_in.at[didx_all.at[j]], sem1, add=True)

    def drain(j):
        pltpu.make_async_copy(ones_v, acc_out.at[sidx_all.at[j]], sem0).wait()
        pltpu.make_async_copy(ones_v, acc_in.at[didx_all.at[j]], sem1).wait()

    fire(0)

    @pl.loop(1, NCHUNK)
    def _(j):
        fire(j)
        drain(j - 1)

    drain(NCHUNK - 1)

    plsc.subcore_barrier()
    obase = c * 2 * NPAD

    pltpu.sync_copy(acc_out.at[pl.ds(r0, RPT)], zrow_v.at[pl.ds(0, RPT)])
    pltpu.sync_copy(zrow_v.at[pl.ds(0, RPT)], out_hbm.at[pl.ds(obase + r0, RPT)])
    pltpu.sync_copy(acc_in.at[pl.ds(r0, RPT)], zrow_v.at[pl.ds(0, RPT)])
    pltpu.sync_copy(zrow_v.at[pl.ds(0, RPT)],
                    out_hbm.at[pl.ds(obase + NPAD + r0, RPT)])


def _degrees(src3, dst3):
    fn = pl.kernel(
        _deg_body,
        out_type=jax.ShapeDtypeStruct((NC * 2 * NPAD,), F32),
        mesh=_MESH,
        scratch_types=[
            pltpu.VMEM((NCHUNK, K), jnp.int32),
            pltpu.VMEM((NCHUNK, K), jnp.int32),
            pltpu.VMEM((K,), F32),
            pltpu.VMEM((RPT + 8,), F32),
            pltpu.VMEM_SHARED((NPAD,), F32),
            pltpu.VMEM_SHARED((NPAD,), F32),
            pltpu.SemaphoreType.DMA,
            pltpu.SemaphoreType.DMA,
        ],
    )
    return fn(src3, dst3)


def _mp_body(h_hbm, src_hbm, dst_hbm, out_hbm, sidx_all,
             didx0, didx1, didx2, rows0, rows1, rows2, acc,
             sg0, sg1, sg2, ss0, ss1, ss2, sd0, sd1, sd2):
    c = lax.axis_index("c")
    s = lax.axis_index("s")
    wid = s * NC + c
    didx = (didx0, didx1, didx2)
    rows = (rows0, rows1, rows2)
    sg = (sg0, sg1, sg2)
    ss = (ss0, ss1, ss2)
    sd = (sd0, sd1, sd2)

    @pl.loop(0, KM)
    def _(r):
        @pl.loop(0, D, step=16)
        def _(c0):
            rows0[r, pl.ds(c0, 16)] = jnp.zeros((16,), F32)

    ebase = wid * EPT
    pltpu.sync_copy(src_hbm.at[pl.ds(ebase, EPT)], sidx_all)

    r0 = s * RPT

    @pl.loop(0, RPT // KM)
    def _(t):
        pltpu.sync_copy(rows0, acc.at[pl.ds(r0 + t * KM, KM)])

    pltpu.sync_copy(rows0.at[pl.ds(0, RPT % KM)],
                    acc.at[pl.ds(r0 + (RPT // KM) * KM, RPT % KM)])
    plsc.subcore_barrier()

    def dload(j, b):
        pltpu.async_copy(dst_hbm.at[pl.ds(ebase + j * KM, KM)], didx[b], sd[b])

    def dwait(j, b):
        pltpu.make_async_copy(dst_hbm.at[pl.ds(ebase + j * KM, KM)],
                              didx[b], sd[b]).wait()

    def gather(j, b):
        pltpu.async_copy(h_hbm.at[sidx_all.at[pl.ds(j * KM, KM)]], rows[b], sg[b])

    def gwait(j, b):
        pltpu.make_async_copy(h_hbm.at[sidx_all.at[pl.ds(j * KM, KM)]],
                              rows[b], sg[b]).wait()

    def scat(b):
        pltpu.async_copy(rows[b], acc.at[didx[b]], ss[b], add=True)

    def swait(b):
        pltpu.make_async_copy(rows[b], acc.at[didx[b]], ss[b]).wait()

    for u in range(3):
        dload(u, u)
        gather(u, u)

    NT = (NCHUNKM - 5) // 3

    @pl.loop(0, NT)
    def _(t):
        j0 = 3 * t
        for u in range(3):
            dwait(j0 + u, u)
            gwait(j0 + u, u)
            scat(u)
        for u in range(3):
            swait(u)
            dload(j0 + 3 + u, u)
            gather(j0 + 3 + u, u)

    jt = 3 * NT
    for u in range(3):
        dwait(jt + u, u)
        gwait(jt + u, u)
        scat(u)
    for u in range(2):
        swait(u)
        dload(jt + 3 + u, u)
        gather(jt + 3 + u, u)
        dwait(jt + 3 + u, u)
        gwait(jt + 3 + u, u)
        scat(u)
    swait(2)
    for u in range(2):
        swait(u)

    plsc.subcore_barrier()
    pltpu.sync_copy(acc.at[pl.ds(r0, RPT)], out_hbm.at[c, pl.ds(r0, RPT)])


def _message_pass(h, src1, dst1):
    fn = pl.kernel(
        _mp_body,
        out_type=jax.ShapeDtypeStruct((NC, NPAD, D), F32),
        mesh=_MESH,
        scratch_types=[
            pltpu.VMEM((EPT,), jnp.int32),
            pltpu.VMEM((KM,), jnp.int32),
            pltpu.VMEM((KM,), jnp.int32),
            pltpu.VMEM((KM,), jnp.int32),
            pltpu.VMEM((KM, D), F32),
            pltpu.VMEM((KM, D), F32),
            pltpu.VMEM((KM, D), F32),
            pltpu.VMEM_SHARED((NPAD, D), F32),
        ] + [pltpu.SemaphoreType.DMA] * 9,
    )
    return fn(h, src1, dst1)


def _ns_of(d_ref):
    return 1.0 / jnp.sqrt(jnp.maximum(d_ref[0, 0] + d_ref[1, 0], 1.0))


def _mm_first(x, degp, w):
    def body(x_ref, d_ref, w_ref, o_ref):
        o_ref[...] = lax.dot_general(
            x_ref[...] * _ns_of(d_ref), w_ref[...],
            (((1,), (0,)), ((), ())), preferred_element_type=F32)

    return pl.pallas_call(
        body,
        grid=(N // BM,),
        in_specs=[
            pl.BlockSpec((BM, x.shape[1]), lambda i: (i, 0)),
            pl.BlockSpec((NC, 2, BM, 1), lambda i: (0, 0, i, 0)),
            pl.BlockSpec(w.shape, lambda i: (0, 0)),
        ],
        out_specs=pl.BlockSpec((BM, w.shape[1]), lambda i: (i, 0)),
        out_shape=jax.ShapeDtypeStruct((NPAD, w.shape[1]), F32),
    )(x, degp, w)


def _mm_mid(p, degp, b, w):
    din = p.shape[2]

    def body(p_ref, d_ref, b_ref, w_ref, o_ref):
        nd = 1.0 / jnp.sqrt(jnp.maximum(d_ref[0, 1] + d_ref[1, 1], 1.0))
        h = (p_ref[0] + p_ref[1]) * nd + b_ref[...]
        h = jnp.maximum(h, 0.0)
        o_ref[...] = lax.dot_general(
            h * _ns_of(d_ref), w_ref[...],
            (((1,), (0,)), ((), ())), preferred_element_type=F32)

    return pl.pallas_call(
        body,
        grid=(N // BM,),
        in_specs=[
            pl.BlockSpec((NC, BM, din), lambda i: (0, i, 0)),
            pl.BlockSpec((NC, 2, BM, 1), lambda i: (0, 0, i, 0)),
            pl.BlockSpec((1, din), lambda i: (0, 0)),
            pl.BlockSpec(w.shape, lambda i: (0, 0)),
        ],
        out_specs=pl.BlockSpec((BM, w.shape[1]), lambda i: (i, 0)),
        out_shape=jax.ShapeDtypeStruct((NPAD, w.shape[1]), F32),
    )(p, degp, b, w)


def _final(p, degp, b):
    dout = p.shape[2]

    def body(p_ref, d_ref, b_ref, o_ref):
        nd = 1.0 / jnp.sqrt(jnp.maximum(d_ref[0, 1] + d_ref[1, 1], 1.0))
        o_ref[...] = (p_ref[0] + p_ref[1]) * nd + b_ref[...]

    return pl.pallas_call(
        body,
        grid=(N // BM,),
        in_specs=[
            pl.BlockSpec((NC, BM, dout), lambda i: (0, i, 0)),
            pl.BlockSpec((NC, 2, BM, 1), lambda i: (0, 0, i, 0)),
            pl.BlockSpec((1, dout), lambda i: (0, 0)),
        ],
        out_specs=pl.BlockSpec((BM, dout), lambda i: (i, 0)),
        out_shape=jax.ShapeDtypeStruct((N, dout), F32),
    )(p, degp, b)


def kernel(features, edge_index, W0, b0, W1, b1, W2, b2):
    ei = edge_index.reshape(2, NW, EPT)
    pad = jnp.broadcast_to((N + jnp.arange(NW, dtype=jnp.int32))[None, :, None],
                           (2, NW, EPTP - EPT))
    eip = jnp.concatenate([ei, pad], axis=2)
    src3 = eip[0].reshape(NW, NCHUNK, K)
    dst3 = eip[1].reshape(NW, NCHUNK, K)
    src1 = edge_index[0]
    dst1 = edge_index[1]

    w2p = jnp.pad(W2, ((0, 0), (0, 88)))
    b2p = jnp.pad(b2, (0, 88))

    degp = _degrees(src3, dst3)
    degp4 = degp.reshape(NC, 2, NPAD, 1)

    h0 = _mm_first(features, degp4, W0)
    p0 = _message_pass(h0, src1, dst1)
    h1 = _mm_mid(p0, degp4, b0.reshape(1, -1), W1)
    p1 = _message_pass(h1, src1, dst1)
    h2 = _mm_mid(p1, degp4, b1.reshape(1, -1), w2p)
    p2 = _message_pass(h2, src1, dst1)
    out = _final(p2, degp4, b2p.reshape(1, -1))
    return out[:, :40]


# speedup vs baseline: 1.0159x; 1.0159x over previous
import jax
import jax.numpy as jnp
from jax import lax
from jax.experimental import pallas as pl
from jax.experimental.pallas import tpu as pltpu
from jax.experimental.pallas import tpu_sc as plsc

N = 10000
E = 320000
NC = 2
NS = 16
NW = NC * NS
K = 128
KM = 80
EPT = E // NW
NCHUNK = 80
NCHUNKM = EPT // KM
EPTP = NCHUNK * K
NPAD = 10112
RPT = NPAD // NS
D = 128
BM = 400
F32 = jnp.float32

_MESH = plsc.VectorSubcoreMesh(core_axis_name="c", subcore_axis_name="s")


def _deg_body(src_hbm, dst_hbm, out_hbm, sidx_all, didx_all, ones_v, zrow_v,
              acc_out, acc_in, sem0, sem1):
    c = lax.axis_index("c")
    s = lax.axis_index("s")
    wid = s * NC + c

    @pl.loop(0, K, step=16)
    def _(i):
        ones_v[pl.ds(i, 16)] = jnp.ones((16,), F32)

    @pl.loop(0, RPT + 8, step=16)
    def _(i):
        zrow_v[pl.ds(i, 16)] = jnp.zeros((16,), F32)

    pltpu.sync_copy(src_hbm.at[wid], sidx_all)
    pltpu.sync_copy(dst_hbm.at[wid], didx_all)

    r0 = s * RPT
    pltpu.sync_copy(zrow_v.at[pl.ds(0, RPT)], acc_out.at[pl.ds(r0, RPT)])
    pltpu.sync_copy(zrow_v.at[pl.ds(0, RPT)], acc_in.at[pl.ds(r0, RPT)])
    plsc.subcore_barrier()

    def fire(j):
        pltpu.async_copy(ones_v, acc_out.at[sidx_all.at[j]], sem0, add=True)
        pltpu.async_copy(ones_v, acc_in.at[didx_all.at[j]], sem1, add=True)

    def drain(j):
        pltpu.make_async_copy(ones_v, acc_out.at[sidx_all.at[j]], sem0).wait()
        pltpu.make_async_copy(ones_v, acc_in.at[didx_all.at[j]], sem1).wait()

    fire(0)

    @pl.loop(1, NCHUNK)
    def _(j):
        fire(j)
        drain(j - 1)

    drain(NCHUNK - 1)

    plsc.subcore_barrier()
    obase = c * 2 * NPAD

    pltpu.sync_copy(acc_out.at[pl.ds(r0, RPT)], zrow_v.at[pl.ds(0, RPT)])
    pltpu.sync_copy(zrow_v.at[pl.ds(0, RPT)], out_hbm.at[pl.ds(obase + r0, RPT)])
    pltpu.sync_copy(acc_in.at[pl.ds(r0, RPT)], zrow_v.at[pl.ds(0, RPT)])
    pltpu.sync_copy(zrow_v.at[pl.ds(0, RPT)],
                    out_hbm.at[pl.ds(obase + NPAD + r0, RPT)])


def _degrees(src3, dst3):
    fn = pl.kernel(
        _deg_body,
        out_type=jax.ShapeDtypeStruct((NC * 2 * NPAD,), F32),
        mesh=_MESH,
        scratch_types=[
            pltpu.VMEM((NCHUNK, K), jnp.int32),
            pltpu.VMEM((NCHUNK, K), jnp.int32),
            pltpu.VMEM((K,), F32),
            pltpu.VMEM((RPT + 8,), F32),
            pltpu.VMEM_SHARED((NPAD,), F32),
            pltpu.VMEM_SHARED((NPAD,), F32),
            pltpu.SemaphoreType.DMA,
            pltpu.SemaphoreType.DMA,
        ],
    )
    return fn(src3, dst3)


def _mp_body(h_hbm, src_hbm, dst_hbm, out_hbm, sidx_all,
             didx0, didx1, didx2, rows0, rows1, rows2, acc,
             sg0, sg1, sg2, ss0, ss1, ss2, sd0, sd1, sd2):
    c = lax.axis_index("c")
    s = lax.axis_index("s")
    wid = s * NC + c
    didx = (didx0, didx1, didx2)
    rows = (rows0, rows1, rows2)
    sg = (sg0, sg1, sg2)
    ss = (ss0, ss1, ss2)
    sd = (sd0, sd1, sd2)

    @pl.loop(0, KM)
    def _(r):
        @pl.loop(0, D, step=16)
        def _(c0):
            rows0[r, pl.ds(c0, 16)] = jnp.zeros((16,), F32)

    ebase = wid * EPT
    pltpu.sync_copy(src_hbm.at[pl.ds(ebase, EPT)], sidx_all)

    r0 = s * RPT

    @pl.loop(0, RPT // KM)
    def _(t):
        pltpu.sync_copy(rows0, acc.at[pl.ds(r0 + t * KM, KM)])

    pltpu.sync_copy(rows0.at[pl.ds(0, RPT % KM)],
                    acc.at[pl.ds(r0 + (RPT // KM) * KM, RPT % KM)])
    plsc.subcore_barrier()

    def dload(j, b):
        pltpu.async_copy(dst_hbm.at[pl.ds(ebase + j * KM, KM)], didx[b], sd[b])

    def dwait(j, b):
        pltpu.make_async_copy(dst_hbm.at[pl.ds(ebase + j * KM, KM)],
                              didx[b], sd[b]).wait()

    def gather(j, b):
        pltpu.async_copy(h_hbm.at[sidx_all.at[pl.ds(j * KM, KM)]], rows[b], sg[b])

    def gwait(j, b):
        pltpu.make_async_copy(h_hbm.at[sidx_all.at[pl.ds(j * KM, KM)]],
                              rows[b], sg[b]).wait()

    def scat(b):
        pltpu.async_copy(rows[b], acc.at[didx[b]], ss[b], add=True)

    def swait(b):
        pltpu.make_async_copy(rows[b], acc.at[didx[b]], ss[b]).wait()

    for u in range(2):
        dload(u, u)
        gather(u, u)

    @pl.loop(0, (NCHUNKM - 3) // 2)
    def _(t):
        j = 2 * t
        for u in range(2):
            dwait(j + u, u)
            gwait(j + u, u)
            pltpu.sync_copy(rows[u], acc.at[didx[u]], add=True)
            dload(j + 2 + u, u)
            gather(j + 2 + u, u)

    jf = NCHUNKM - 3
    for u in range(2):
        dwait(jf + u, u)
        gwait(jf + u, u)
        pltpu.sync_copy(rows[u], acc.at[didx[u]], add=True)
        if u == 0:
            dload(jf + 2, 0)
            gather(jf + 2, 0)
    dwait(jf + 2, 0)
    gwait(jf + 2, 0)
    pltpu.sync_copy(rows[0], acc.at[didx[0]], add=True)

    plsc.subcore_barrier()
    pltpu.sync_copy(acc.at[pl.ds(r0, RPT)], out_hbm.at[c, pl.ds(r0, RPT)])


def _message_pass(h, src1, dst1):
    fn = pl.kernel(
        _mp_body,
        out_type=jax.ShapeDtypeStruct((NC, NPAD, D), F32),
        mesh=_MESH,
        scratch_types=[
            pltpu.VMEM((EPT,), jnp.int32),
            pltpu.VMEM((KM,), jnp.int32),
            pltpu.VMEM((KM,), jnp.int32),
            pltpu.VMEM((KM,), jnp.int32),
            pltpu.VMEM((KM, D), F32),
            pltpu.VMEM((KM, D), F32),
            pltpu.VMEM((KM, D), F32),
            pltpu.VMEM_SHARED((NPAD, D), F32),
        ] + [pltpu.SemaphoreType.DMA] * 9,
    )
    return fn(h, src1, dst1)


def _ns_of(d_ref):
    return 1.0 / jnp.sqrt(jnp.maximum(d_ref[0, 0] + d_ref[1, 0], 1.0))


def _mm_first(x, degp, w):
    def body(x_ref, d_ref, w_ref, o_ref):
        o_ref[...] = lax.dot_general(
            x_ref[...] * _ns_of(d_ref), w_ref[...],
            (((1,), (0,)), ((), ())), preferred_element_type=F32)

    return pl.pallas_call(
        body,
        grid=(N // BM,),
        in_specs=[
            pl.BlockSpec((BM, x.shape[1]), lambda i: (i, 0)),
            pl.BlockSpec((NC, 2, BM, 1), lambda i: (0, 0, i, 0)),
            pl.BlockSpec(w.shape, lambda i: (0, 0)),
        ],
        out_specs=pl.BlockSpec((BM, w.shape[1]), lambda i: (i, 0)),
        out_shape=jax.ShapeDtypeStruct((NPAD, w.shape[1]), F32),
    )(x, degp, w)


def _mm_mid(p, degp, b, w):
    din = p.shape[2]

    def body(p_ref, d_ref, b_ref, w_ref, o_ref):
        nd = 1.0 / jnp.sqrt(jnp.maximum(d_ref[0, 1] + d_ref[1, 1], 1.0))
        h = (p_ref[0] + p_ref[1]) * nd + b_ref[...]
        h = jnp.maximum(h, 0.0)
        o_ref[...] = lax.dot_general(
            h * _ns_of(d_ref), w_ref[...],
            (((1,), (0,)), ((), ())), preferred_element_type=F32)

    return pl.pallas_call(
        body,
        grid=(N // BM,),
        in_specs=[
            pl.BlockSpec((NC, BM, din), lambda i: (0, i, 0)),
            pl.BlockSpec((NC, 2, BM, 1), lambda i: (0, 0, i, 0)),
            pl.BlockSpec((1, din), lambda i: (0, 0)),
            pl.BlockSpec(w.shape, lambda i: (0, 0)),
        ],
        out_specs=pl.BlockSpec((BM, w.shape[1]), lambda i: (i, 0)),
        out_shape=jax.ShapeDtypeStruct((NPAD, w.shape[1]), F32),
    )(p, degp, b, w)


def _final(p, degp, b):
    dout = p.shape[2]

    def body(p_ref, d_ref, b_ref, o_ref):
        nd = 1.0 / jnp.sqrt(jnp.maximum(d_ref[0, 1] + d_ref[1, 1], 1.0))
        o_ref[...] = (p_ref[0] + p_ref[1]) * nd + b_ref[...]

    return pl.pallas_call(
        body,
        grid=(N // BM,),
        in_specs=[
            pl.BlockSpec((NC, BM, dout), lambda i: (0, i, 0)),
            pl.BlockSpec((NC, 2, BM, 1), lambda i: (0, 0, i, 0)),
            pl.BlockSpec((1, dout), lambda i: (0, 0)),
        ],
        out_specs=pl.BlockSpec((BM, dout), lambda i: (i, 0)),
        out_shape=jax.ShapeDtypeStruct((N, dout), F32),
    )(p, degp, b)


def kernel(features, edge_index, W0, b0, W1, b1, W2, b2):
    ei = edge_index.reshape(2, NW, EPT)
    pad = jnp.broadcast_to((N + jnp.arange(NW, dtype=jnp.int32))[None, :, None],
                           (2, NW, EPTP - EPT))
    eip = jnp.concatenate([ei, pad], axis=2)
    src3 = eip[0].reshape(NW, NCHUNK, K)
    dst3 = eip[1].reshape(NW, NCHUNK, K)
    src1 = edge_index[0]
    dst1 = edge_index[1]

    w2p = jnp.pad(W2, ((0, 0), (0, 88)))
    b2p = jnp.pad(b2, (0, 88))

    degp = _degrees(src3, dst3)
    degp4 = degp.reshape(NC, 2, NPAD, 1)

    h0 = _mm_first(features, degp4, W0)
    p0 = _message_pass(h0, src1, dst1)
    h1 = _mm_mid(p0, degp4, b0.reshape(1, -1), W1)
    p1 = _message_pass(h1, src1, dst1)
    h2 = _mm_mid(p1, degp4, b1.reshape(1, -1), w2p)
    p2 = _message_pass(h2, src1, dst1)
    out = _final(p2, degp4, b2p.reshape(1, -1))
    return out[:, :40]


# speedup vs baseline: 1.0284x; 1.0124x over previous
import jax
import jax.numpy as jnp
from jax import lax
from jax.experimental import pallas as pl
from jax.experimental.pallas import tpu as pltpu
from jax.experimental.pallas import tpu_sc as plsc

N = 10000
E = 320000
NC = 2
NS = 16
NW = NC * NS
K = 128
KM = 80
EPT = E // NW
NCHUNK = 80
NCHUNKM = EPT // KM
EPTP = NCHUNK * K
NPAD = 10112
RPT = NPAD // NS
D = 128
BM = 400
F32 = jnp.float32

_MESH = plsc.VectorSubcoreMesh(core_axis_name="c", subcore_axis_name="s")


def _deg_body(src_hbm, dst_hbm, out_hbm, sidx_all, didx_all, ones_v, zrow_v,
              acc_out, acc_in, sem0, sem1):
    c = lax.axis_index("c")
    s = lax.axis_index("s")
    wid = s * NC + c

    pltpu.async_copy(src_hbm.at[wid], sidx_all, sem0)
    pltpu.async_copy(dst_hbm.at[wid], didx_all, sem1)

    @pl.loop(0, K, step=16)
    def _(i):
        ones_v[pl.ds(i, 16)] = jnp.ones((16,), F32)

    @pl.loop(0, RPT + 8, step=16)
    def _(i):
        zrow_v[pl.ds(i, 16)] = jnp.zeros((16,), F32)

    pltpu.make_async_copy(src_hbm.at[wid], sidx_all, sem0).wait()
    pltpu.make_async_copy(dst_hbm.at[wid], didx_all, sem1).wait()

    r0 = s * RPT
    pltpu.sync_copy(zrow_v.at[pl.ds(0, RPT)], acc_out.at[pl.ds(r0, RPT)])
    pltpu.sync_copy(zrow_v.at[pl.ds(0, RPT)], acc_in.at[pl.ds(r0, RPT)])
    plsc.subcore_barrier()

    def fire(j):
        pltpu.async_copy(ones_v, acc_out.at[sidx_all.at[j]], sem0, add=True)
        pltpu.async_copy(ones_v, acc_in.at[didx_all.at[j]], sem1, add=True)

    def drain(j):
        pltpu.make_async_copy(ones_v, acc_out.at[sidx_all.at[j]], sem0).wait()
        pltpu.make_async_copy(ones_v, acc_in.at[didx_all.at[j]], sem1).wait()

    fire(0)

    @pl.loop(1, NCHUNK)
    def _(j):
        fire(j)
        drain(j - 1)

    drain(NCHUNK - 1)

    plsc.subcore_barrier()
    obase = c * 2 * NPAD

    pltpu.sync_copy(acc_out.at[pl.ds(r0, RPT)], zrow_v.at[pl.ds(0, RPT)])
    pltpu.sync_copy(zrow_v.at[pl.ds(0, RPT)], out_hbm.at[pl.ds(obase + r0, RPT)])
    pltpu.sync_copy(acc_in.at[pl.ds(r0, RPT)], zrow_v.at[pl.ds(0, RPT)])
    pltpu.sync_copy(zrow_v.at[pl.ds(0, RPT)],
                    out_hbm.at[pl.ds(obase + NPAD + r0, RPT)])


def _degrees(src3, dst3):
    fn = pl.kernel(
        _deg_body,
        out_type=jax.ShapeDtypeStruct((NC * 2 * NPAD,), F32),
        mesh=_MESH,
        scratch_types=[
            pltpu.VMEM((NCHUNK, K), jnp.int32),
            pltpu.VMEM((NCHUNK, K), jnp.int32),
            pltpu.VMEM((K,), F32),
            pltpu.VMEM((RPT + 8,), F32),
            pltpu.VMEM_SHARED((NPAD,), F32),
            pltpu.VMEM_SHARED((NPAD,), F32),
            pltpu.SemaphoreType.DMA,
            pltpu.SemaphoreType.DMA,
        ],
    )
    return fn(src3, dst3)


def _mp_body(h_hbm, src_hbm, dst_hbm, out_hbm, sidx_all,
             didx0, didx1, didx2, rows0, rows1, rows2, acc,
             sg0, sg1, sg2, ss0, ss1, ss2, sd0, sd1, sd2):
    c = lax.axis_index("c")
    s = lax.axis_index("s")
    wid = s * NC + c
    didx = (didx0, didx1, didx2)
    rows = (rows0, rows1, rows2)
    sg = (sg0, sg1, sg2)
    ss = (ss0, ss1, ss2)
    sd = (sd0, sd1, sd2)

    ebase = wid * EPT
    r0 = s * RPT

    pltpu.async_copy(src_hbm.at[pl.ds(ebase, EPT)], sidx_all, sg2)

    @pl.loop(0, KM)
    def _(r):
        @pl.loop(0, D, step=16)
        def _(c0):
            rows2[r, pl.ds(c0, 16)] = jnp.zeros((16,), F32)

    pltpu.make_async_copy(src_hbm.at[pl.ds(ebase, EPT)], sidx_all, sg2).wait()

    def dload(j, b):
        pltpu.async_copy(dst_hbm.at[pl.ds(ebase + j * KM, KM)], didx[b], sd[b])

    def dwait(j, b):
        pltpu.make_async_copy(dst_hbm.at[pl.ds(ebase + j * KM, KM)],
                              didx[b], sd[b]).wait()

    def gather(j, b):
        pltpu.async_copy(h_hbm.at[sidx_all.at[pl.ds(j * KM, KM)]], rows[b], sg[b])

    def gwait(j, b):
        pltpu.make_async_copy(h_hbm.at[sidx_all.at[pl.ds(j * KM, KM)]],
                              rows[b], sg[b]).wait()

    def scat(b):
        pltpu.async_copy(rows[b], acc.at[didx[b]], ss[b], add=True)

    def swait(b):
        pltpu.make_async_copy(rows[b], acc.at[didx[b]], ss[b]).wait()

    for u in range(2):
        dload(u, u)
        gather(u, u)

    @pl.loop(0, RPT // KM)
    def _(t):
        pltpu.sync_copy(rows2, acc.at[pl.ds(r0 + t * KM, KM)])

    pltpu.sync_copy(rows2.at[pl.ds(0, RPT % KM)],
                    acc.at[pl.ds(r0 + (RPT // KM) * KM, RPT % KM)])
    plsc.subcore_barrier()

    @pl.loop(0, (NCHUNKM - 3) // 2)
    def _(t):
        j = 2 * t
        for u in range(2):
            dwait(j + u, u)
            gwait(j + u, u)
            pltpu.sync_copy(rows[u], acc.at[didx[u]], add=True)
            dload(j + 2 + u, u)
            gather(j + 2 + u, u)

    jf = NCHUNKM - 3
    for u in range(2):
        dwait(jf + u, u)
        gwait(jf + u, u)
        pltpu.sync_copy(rows[u], acc.at[didx[u]], add=True)
        if u == 0:
            dload(jf + 2, 0)
            gather(jf + 2, 0)
    dwait(jf + 2, 0)
    gwait(jf + 2, 0)
    pltpu.sync_copy(rows[0], acc.at[didx[0]], add=True)

    plsc.subcore_barrier()
    pltpu.sync_copy(acc.at[pl.ds(r0, RPT)], out_hbm.at[c, pl.ds(r0, RPT)])


def _message_pass(h, src1, dst1):
    fn = pl.kernel(
        _mp_body,
        out_type=jax.ShapeDtypeStruct((NC, NPAD, D), F32),
        mesh=_MESH,
        scratch_types=[
            pltpu.VMEM((EPT,), jnp.int32),
            pltpu.VMEM((KM,), jnp.int32),
            pltpu.VMEM((KM,), jnp.int32),
            pltpu.VMEM((KM,), jnp.int32),
            pltpu.VMEM((KM, D), F32),
            pltpu.VMEM((KM, D), F32),
            pltpu.VMEM((KM, D), F32),
            pltpu.VMEM_SHARED((NPAD, D), F32),
        ] + [pltpu.SemaphoreType.DMA] * 9,
    )
    return fn(h, src1, dst1)


def _ns_of(d_ref):
    return 1.0 / jnp.sqrt(jnp.maximum(d_ref[0, 0] + d_ref[1, 0], 1.0))


def _mm_first(x, degp, w):
    def body(x_ref, d_ref, w_ref, o_ref):
        o_ref[...] = lax.dot_general(
            x_ref[...] * _ns_of(d_ref), w_ref[...],
            (((1,), (0,)), ((), ())), preferred_element_type=F32)

    return pl.pallas_call(
        body,
        grid=(N // BM,),
        in_specs=[
            pl.BlockSpec((BM, x.shape[1]), lambda i: (i, 0)),
            pl.BlockSpec((NC, 2, BM, 1), lambda i: (0, 0, i, 0)),
            pl.BlockSpec(w.shape, lambda i: (0, 0)),
        ],
        out_specs=pl.BlockSpec((BM, w.shape[1]), lambda i: (i, 0)),
        out_shape=jax.ShapeDtypeStruct((NPAD, w.shape[1]), F32),
    )(x, degp, w)


def _mm_mid(p, degp, b, w):
    din = p.shape[2]

    def body(p_ref, d_ref, b_ref, w_ref, o_ref):
        nd = 1.0 / jnp.sqrt(jnp.maximum(d_ref[0, 1] + d_ref[1, 1], 1.0))
        h = (p_ref[0] + p_ref[1]) * nd + b_ref[...]
        h = jnp.maximum(h, 0.0)
        o_ref[...] = lax.dot_general(
            h * _ns_of(d_ref), w_ref[...],
            (((1,), (0,)), ((), ())), preferred_element_type=F32)

    return pl.pallas_call(
        body,
        grid=(N // BM,),
        in_specs=[
            pl.BlockSpec((NC, BM, din), lambda i: (0, i, 0)),
            pl.BlockSpec((NC, 2, BM, 1), lambda i: (0, 0, i, 0)),
            pl.BlockSpec((1, din), lambda i: (0, 0)),
            pl.BlockSpec(w.shape, lambda i: (0, 0)),
        ],
        out_specs=pl.BlockSpec((BM, w.shape[1]), lambda i: (i, 0)),
        out_shape=jax.ShapeDtypeStruct((NPAD, w.shape[1]), F32),
    )(p, degp, b, w)


def _final(p, degp, b):
    dout = p.shape[2]

    def body(p_ref, d_ref, b_ref, o_ref):
        nd = 1.0 / jnp.sqrt(jnp.maximum(d_ref[0, 1] + d_ref[1, 1], 1.0))
        v = (p_ref[0] + p_ref[1]) * nd + b_ref[...]
        o_ref[...] = v[:, :40]

    return pl.pallas_call(
        body,
        grid=(N // BM,),
        in_specs=[
            pl.BlockSpec((NC, BM, dout), lambda i: (0, i, 0)),
            pl.BlockSpec((NC, 2, BM, 1), lambda i: (0, 0, i, 0)),
            pl.BlockSpec((1, dout), lambda i: (0, 0)),
        ],
        out_specs=pl.BlockSpec((BM, 40), lambda i: (i, 0)),
        out_shape=jax.ShapeDtypeStruct((N, 40), F32),
    )(p, degp, b)


def kernel(features, edge_index, W0, b0, W1, b1, W2, b2):
    ei = edge_index.reshape(2, NW, EPT)
    pad = jnp.broadcast_to((N + jnp.arange(NW, dtype=jnp.int32))[None, :, None],
                           (2, NW, EPTP - EPT))
    eip = jnp.concatenate([ei, pad], axis=2)
    src3 = eip[0].reshape(NW, NCHUNK, K)
    dst3 = eip[1].reshape(NW, NCHUNK, K)
    src1 = edge_index[0]
    dst1 = edge_index[1]

    w2p = jnp.pad(W2, ((0, 0), (0, 88)))
    b2p = jnp.pad(b2, (0, 88))

    degp = _degrees(src3, dst3)
    degp4 = degp.reshape(NC, 2, NPAD, 1)

    h0 = _mm_first(features, degp4, W0)
    p0 = _message_pass(h0, src1, dst1)
    h1 = _mm_mid(p0, degp4, b0.reshape(1, -1), W1)
    p1 = _message_pass(h1, src1, dst1)
    h2 = _mm_mid(p1, degp4, b1.reshape(1, -1), w2p)
    p2 = _message_pass(h2, src1, dst1)
    return _final(p2, degp4, b2p.reshape(1, -1))
